# baseline (speedup 1.0000x reference)
import jax
import jax.numpy as jnp
from jax.experimental import pallas as pl
from jax.experimental.pallas import tpu as pltpu

_T = 0.05
_CHUNKS = [24576, 24576, 24576, 24576, 1664]
_ALIGNED = 99968


def _matmul_kernel(x_ref, memt_ref, out_ref):
    acc = jax.lax.dot_general(
        x_ref[...],
        memt_ref[...],
        dimension_numbers=(((1,), (0,)), ((), ())),
        preferred_element_type=jnp.float32,
    )
    out_ref[...] = acc / _T


def _chunk_call(x, memt_chunk, width, bn):
    m, k = x.shape
    return pl.pallas_call(
        _matmul_kernel,
        grid=(width // bn,),
        in_specs=[
            pl.BlockSpec((m, k), lambda i: (0, 0)),
            pl.BlockSpec((k, bn), lambda i: (0, i)),
        ],
        out_specs=pl.BlockSpec((m, bn), lambda i: (0, i)),
        out_shape=jax.ShapeDtypeStruct((m, width), jnp.float32),
        compiler_params=pltpu.CompilerParams(
            dimension_semantics=("arbitrary",),
            vmem_limit_bytes=63 * 1024 * 1024,
        ),
    )(x, memt_chunk)


def _tail_call(x, memt, n):
    m, k = x.shape
    blk = _ALIGNED // 128
    return pl.pallas_call(
        _matmul_kernel,
        grid=(1,),
        in_specs=[
            pl.BlockSpec((m, k), lambda i: (0, 0)),
            pl.BlockSpec((k, 128), lambda i: (0, blk)),
        ],
        out_specs=pl.BlockSpec((m, 128), lambda i: (0, blk)),
        out_shape=jax.ShapeDtypeStruct((m, n), jnp.float32),
        compiler_params=pltpu.CompilerParams(
            dimension_semantics=("arbitrary",),
            vmem_limit_bytes=63 * 1024 * 1024,
        ),
    )(x, memt)


@jax.jit
def kernel(x, memory):
    n = memory.shape[0]
    memt = memory.T
    buf = _tail_call(x, memt, n)
    off = 0
    for w in _CHUNKS:
        bn = 4096 if w % 4096 == 0 else w
        mc = jax.lax.slice_in_dim(memt, off, off + w, axis=1)
        c = _chunk_call(x, mc, w, bn)
        buf = jax.lax.dynamic_update_slice(buf, c, (0, off))
        off += w
    return buf
